# Initial kernel scaffold; baseline (speedup 1.0000x reference)
import jax
import jax.numpy as jnp
from jax.experimental import pallas as pl


def kernel(inp, Wg, bg, W1, W2, ln_gamma, ln_beta, bias):
    raise NotImplementedError("write your pallas kernel here")


# speedup vs baseline: 5.3689x; 5.3689x over previous
import functools

import jax
import jax.numpy as jnp
from jax import lax
from jax.experimental import pallas as pl
from jax.experimental.pallas import tpu as pltpu
from jax.experimental.pallas import tpu_sc as plsc

E = 8
D = 768
H = 3072
K = 2
N = 2048
TM = 128
P = N * K + E * TM
NT = P // TM


def _gate_body(x_ref, wg_ref, bg_ref, e_ref, w_ref):
    x = x_ref[...]
    logits = lax.dot_general(x, wg_ref[...], (((1,), (1,)), ((), ())),
                             precision=lax.Precision.DEFAULT,
                             preferred_element_type=jnp.float32)
    logits = logits + bg_ref[...]
    iot = lax.broadcasted_iota(jnp.int32, (N, E), 1)
    m0 = jnp.max(logits, axis=1, keepdims=True)
    e0 = jnp.min(jnp.where(logits == m0, iot, E), axis=1, keepdims=True)
    l1 = jnp.where(iot == e0, jnp.float32(-1e30), logits)
    m1 = jnp.max(l1, axis=1, keepdims=True)
    e1 = jnp.min(jnp.where(l1 == m1, iot, E), axis=1, keepdims=True)
    t = jnp.exp(m1 - m0)
    w0 = 1.0 / (1.0 + t)
    e_ref[...] = jnp.concatenate([e0, e1], axis=1)
    w_ref[...] = jnp.concatenate([w0, 1.0 - w0], axis=1)


def _gate(x, Wg, bg):
    return pl.pallas_call(
        _gate_body,
        out_shape=(jax.ShapeDtypeStruct((N, K), jnp.int32),
                   jax.ShapeDtypeStruct((N, K), jnp.float32)),
    )(x, Wg, bg.reshape(1, E))


def _sc_dispatch(x, dest_pair):
    info = plsc.get_sparse_core_info()
    nw = info.num_cores * info.num_subcores
    tpw = N // nw

    @functools.partial(
        pl.kernel,
        mesh=plsc.VectorSubcoreMesh(core_axis_name="c", subcore_axis_name="s"),
        out_type=jax.ShapeDtypeStruct((P, D), jnp.float32),
        scratch_types=[pltpu.VMEM((tpw, D), jnp.float32),
                       pltpu.VMEM((tpw,), jnp.int32),
                       pltpu.SemaphoreType.DMA],
    )
    def k(x_hbm, dp_hbm, xs_hbm, xv, idxv, sem):
        wid = lax.axis_index("s") * info.num_cores + lax.axis_index("c")
        base = wid * tpw
        pltpu.sync_copy(x_hbm.at[pl.ds(base, tpw)], xv)
        for kk in range(K):
            pltpu.sync_copy(dp_hbm.at[kk, pl.ds(base, tpw)], idxv)
            pltpu.async_copy(xv, xs_hbm.at[idxv], sem).wait()

    return k(x, dest_pair)


def _sc_combine(ys, dest_pair):
    info = plsc.get_sparse_core_info()
    nw = info.num_cores * info.num_subcores
    tpw = N // nw

    @functools.partial(
        pl.kernel,
        mesh=plsc.VectorSubcoreMesh(core_axis_name="c", subcore_axis_name="s"),
        out_type=jax.ShapeDtypeStruct((K, N, D), jnp.float32),
        scratch_types=[pltpu.VMEM((tpw, D), jnp.float32),
                       pltpu.VMEM((tpw,), jnp.int32),
                       pltpu.SemaphoreType.DMA],
    )
    def k(ys_hbm, dp_hbm, yp_hbm, yv, idxv, sem):
        wid = lax.axis_index("s") * info.num_cores + lax.axis_index("c")
        base = wid * tpw
        for kk in range(K):
            pltpu.sync_copy(dp_hbm.at[kk, pl.ds(base, tpw)], idxv)
            pltpu.async_copy(ys_hbm.at[idxv], yv, sem).wait()
            pltpu.sync_copy(yv, yp_hbm.at[kk, pl.ds(base, tpw)])

    return k(ys, dest_pair)


def _mlp_body(te_ref, x_ref, w1_ref, w2_ref, y_ref):
    x = x_ref[...]
    h = lax.dot_general(x, w1_ref[0], (((1,), (1,)), ((), ())),
                        preferred_element_type=jnp.float32)
    h = 0.5 * h * (1.0 + lax.erf(h * 0.7071067811865476))
    y_ref[...] = lax.dot_general(h, w2_ref[0], (((1,), (1,)), ((), ())),
                                 preferred_element_type=jnp.float32)


def _mlp(x_sorted, W1, W2, tile_expert):
    grid_spec = pltpu.PrefetchScalarGridSpec(
        num_scalar_prefetch=1,
        grid=(NT,),
        in_specs=[
            pl.BlockSpec((TM, D), lambda i, s: (i, 0)),
            pl.BlockSpec((1, H, D), lambda i, s: (s[i], 0, 0)),
            pl.BlockSpec((1, D, H), lambda i, s: (s[i], 0, 0)),
        ],
        out_specs=pl.BlockSpec((TM, D), lambda i, s: (i, 0)),
    )
    return pl.pallas_call(
        _mlp_body,
        grid_spec=grid_spec,
        out_shape=jax.ShapeDtypeStruct((P, D), jnp.float32),
        compiler_params=pltpu.CompilerParams(
            dimension_semantics=("arbitrary",)),
    )(tile_expert, x_sorted, W1, W2)


def _combine_body(yp_ref, w_ref, x_ref, g_ref, b_ref, o_ref):
    w = w_ref[...]
    y = w[:, 0:1] * yp_ref[0] + w[:, 1:2] * yp_ref[1] + x_ref[...]
    mu = jnp.mean(y, axis=1, keepdims=True)
    yc = y - mu
    var = jnp.mean(yc * yc, axis=1, keepdims=True)
    o_ref[...] = yc * lax.rsqrt(var + 1e-5) * g_ref[...] + b_ref[...]


def _combine(yp, w, x, gamma, beta):
    return pl.pallas_call(
        _combine_body,
        out_shape=jax.ShapeDtypeStruct((N, D), jnp.float32),
    )(yp, w, x, gamma.reshape(1, D), beta.reshape(1, D))


def _route_metadata(e2):
    ef = e2.reshape(-1)
    oh = (ef[:, None] == jnp.arange(E, dtype=jnp.int32)).astype(jnp.int32)
    csum = jnp.cumsum(oh, axis=0)
    rank = jnp.take_along_axis(csum, ef[:, None], axis=1)[:, 0] - 1
    counts = csum[-1]
    padded = ((counts + TM - 1) // TM) * TM
    al_off = jnp.concatenate(
        [jnp.zeros((1,), jnp.int32), jnp.cumsum(padded)[:-1]])
    dest = al_off[ef] + rank
    dest_pair = dest.reshape(N, K).T
    tile_ends = jnp.cumsum(padded // TM)
    tile_expert = jnp.clip(
        jnp.searchsorted(tile_ends, jnp.arange(NT), side="right"),
        0, E - 1).astype(jnp.int32)
    return dest_pair, tile_expert


def kernel(inp, Wg, bg, W1, W2, ln_gamma, ln_beta, bias):
    x = inp.reshape(N, D)
    e2, gw = _gate(x, Wg, bg)
    dest_pair, tile_expert = _route_metadata(e2)
    xs = _sc_dispatch(x, dest_pair)
    ys = _mlp(xs, W1, W2, tile_expert)
    yp = _sc_combine(ys, dest_pair)
    out = _combine(yp, gw, x, ln_gamma, ln_beta)
    return (out.reshape(inp.shape), bias)
